# baseline (speedup 1.0000x reference)
import functools

import jax
import jax.numpy as jnp
from jax import lax
from jax.experimental import pallas as pl
from jax.experimental.pallas import tpu as pltpu
from jax.experimental.pallas import tpu_sc as plsc

B = 8
L = 256
C = 8
D = 256
H = 256
NB = 2 * B
N_NODES = L * NB
N_ROWS = 9 * N_NODES
N_WORKERS = 32
ROWS_PER_W = N_ROWS // N_WORKERS
GCHUNK = 128
N_CHUNKS = ROWS_PER_W // GCHUNK
_BLKN = 128
_NBLK = N_NODES // _BLKN


@functools.lru_cache(maxsize=1)
def _sc_gather_fn():
    mesh = plsc.VectorSubcoreMesh(core_axis_name="c", subcore_axis_name="s",
                                  num_cores=2)

    @functools.partial(
        pl.kernel,
        out_type=jax.ShapeDtypeStruct((N_ROWS, D), jnp.float32),
        mesh=mesh,
        scratch_types=[
            pltpu.VMEM((GCHUNK,), jnp.int32),
            pltpu.VMEM((GCHUNK, D), jnp.float32),
            pltpu.SemaphoreType.DMA,
        ],
    )
    def gather(idx_hbm, table_hbm, out_hbm, idx_v, rows_v, sem):
        wid = lax.axis_index("s") * 2 + lax.axis_index("c")
        base = wid * ROWS_PER_W

        def chunk(i, carry):
            off = base + i * GCHUNK
            pltpu.sync_copy(idx_hbm.at[pl.ds(off, GCHUNK)], idx_v)
            pltpu.async_copy(table_hbm.at[idx_v], rows_v, sem).wait()
            pltpu.sync_copy(rows_v, out_hbm.at[pl.ds(off, GCHUNK)])
            return carry

        lax.fori_loop(0, N_CHUNKS, chunk, 0)

    return gather


def _sc_gather(idx, table):
    return _sc_gather_fn()(idx, table)


def _fused_body(rows_ref, wn_ref, bn_ref,
                wif0_ref, whf0_ref, bf0_ref, wib0_ref, whb0_ref, bb0_ref,
                wif1_ref, whf1_ref, bf1_ref, wib1_ref, whb1_ref, bb1_ref,
                wc_ref, bc_ref, w2_ref, b2_ref,
                out_ref, seq_buf, gif_ref, gib_ref, h_buf):
    i = pl.program_id(0)

    @pl.when(i < _NBLK)
    def node_phase():
        x = rows_ref[...]
        y = jnp.dot(x.reshape(9 * _BLKN, D), wn_ref[...],
                    preferred_element_type=jnp.float32) + bn_ref[...]
        y = y.reshape(9, _BLKN, D)
        er = y[0]
        maxc = jnp.max(y[1:], axis=0)
        sumc = jnp.sum(y[1:], axis=0)
        blk = pl.multiple_of(i * _BLKN, _BLKN)
        seq_buf[pl.ds(blk, _BLKN), :] = jnp.maximum(
            jnp.maximum(0.0, maxc), er + sumc)

    @pl.when(i == _NBLK)
    def gru_phase():
        def cell(h, gi, gh):
            r = jax.nn.sigmoid(gi[:, 0:H] + gh[:, 0:H])
            z = jax.nn.sigmoid(gi[:, H:2 * H] + gh[:, H:2 * H])
            n = jnp.tanh(gi[:, 2 * H:3 * H] + r * gh[:, 2 * H:3 * H])
            return (1.0 - z) * n + z * h

        def bigru(x, wif_ref, bf_ref, wib_ref, bb_ref, whf_ref, whb_ref):
            gif_ref[...] = jnp.dot(x, wif_ref[...],
                                   preferred_element_type=jnp.float32) + bf_ref[...]
            gib_ref[...] = jnp.dot(x, wib_ref[...],
                                   preferred_element_type=jnp.float32) + bb_ref[...]
            whf = whf_ref[...]
            whb = whb_ref[...]

            def step(t, carry):
                h_f, h_b = carry
                tf = pl.multiple_of(t * NB, NB)
                tb = pl.multiple_of((L - 1 - t) * NB, NB)
                gf = gif_ref[pl.ds(tf, NB), :]
                gb = gib_ref[pl.ds(tb, NB), :]
                gh_f = jnp.dot(h_f, whf, preferred_element_type=jnp.float32)
                gh_b = jnp.dot(h_b, whb, preferred_element_type=jnp.float32)
                h_f = cell(h_f, gf, gh_f)
                h_b = cell(h_b, gb, gh_b)
                h_buf[pl.ds(tf, NB), 0:H] = h_f
                h_buf[pl.ds(tb, NB), H:2 * H] = h_b
                return (h_f, h_b)

            h0 = jnp.zeros((NB, H), jnp.float32)
            lax.fori_loop(0, L, step, (h0, h0))

        bigru(seq_buf[...], wif0_ref, bf0_ref, wib0_ref, bb0_ref,
              whf0_ref, whb0_ref)
        bigru(h_buf[...], wif1_ref, bf1_ref, wib1_ref, bb1_ref,
              whf1_ref, whb1_ref)

        seq_buf[...] = jnp.dot(h_buf[...], wc_ref[...],
                               preferred_element_type=jnp.float32) + bc_ref[...]

        def red(t, carry):
            m, s = carry
            tt = pl.multiple_of(t * NB, NB)
            blk = seq_buf[pl.ds(tt, NB), :]
            return jnp.maximum(m, blk), s + blk

        init = seq_buf[pl.ds(0, NB), :]
        m, s = lax.fori_loop(1, L, red, (init, init))
        z1 = jnp.sum(m[0:B] * m[B:2 * B], axis=1, keepdims=True)
        z2 = jnp.sum(s[0:B] * s[B:2 * B] * w2_ref[...], axis=1,
                     keepdims=True) + b2_ref[0, 0]
        out_ref[...] = z1 + z2


def _fused_call(rows3, wn, bn, l0, l1, wc, bc, w2, b2):
    full = lambda shape: pl.BlockSpec(shape, lambda i: tuple(0 for _ in shape))
    return pl.pallas_call(
        _fused_body,
        grid=(_NBLK + 1,),
        in_specs=[
            pl.BlockSpec((9, _BLKN, D), lambda i: (0, jnp.minimum(i, _NBLK - 1), 0)),
            full((D, D)), full((1, D)),
            full((D, 3 * H)), full((H, 3 * H)), full((1, 3 * H)),
            full((D, 3 * H)), full((H, 3 * H)), full((1, 3 * H)),
            full((2 * H, 3 * H)), full((H, 3 * H)), full((1, 3 * H)),
            full((2 * H, 3 * H)), full((H, 3 * H)), full((1, 3 * H)),
            full((2 * H, H)), full((1, H)), full((1, H)), full((1, 1)),
        ],
        out_specs=pl.BlockSpec((B, 1), lambda i: (0, 0)),
        out_shape=jax.ShapeDtypeStruct((B, 1), jnp.float32),
        scratch_shapes=[
            pltpu.VMEM((N_NODES, D), jnp.float32),
            pltpu.VMEM((N_NODES, 3 * H), jnp.float32),
            pltpu.VMEM((N_NODES, 3 * H), jnp.float32),
            pltpu.VMEM((N_NODES, 2 * H), jnp.float32),
        ],
    )(rows3, wn, bn, *l0, *l1, wc, bc, w2, b2)


def kernel(root1, child1, root2, child2, embed, W_lin, b_lin,
           Wih_l0_f, Whh_l0_f, bih_l0_f, bhh_l0_f,
           Wih_l0_b, Whh_l0_b, bih_l0_b, bhh_l0_b,
           Wih_l1_f, Whh_l1_f, bih_l1_f, bhh_l1_f,
           Wih_l1_b, Whh_l1_b, bih_l1_b, bhh_l1_b,
           W_comb, b_comb, W2, b2):
    root = jnp.stack([root1, root2])
    child = jnp.stack([child1, child2])
    root_t = root.transpose(2, 0, 1).reshape(1, L, NB)
    child_t = child.transpose(3, 2, 0, 1).reshape(C, L, NB)
    idx = jnp.concatenate([root_t, child_t], axis=0).reshape(-1)
    idx = idx.astype(jnp.int32)

    rows = _sc_gather(idx, embed)
    rows3 = rows.reshape(9, N_NODES, D)

    l0 = (Wih_l0_f.T, Whh_l0_f.T, (bih_l0_f + bhh_l0_f).reshape(1, -1),
          Wih_l0_b.T, Whh_l0_b.T, (bih_l0_b + bhh_l0_b).reshape(1, -1))
    l1 = (Wih_l1_f.T, Whh_l1_f.T, (bih_l1_f + bhh_l1_f).reshape(1, -1),
          Wih_l1_b.T, Whh_l1_b.T, (bih_l1_b + bhh_l1_b).reshape(1, -1))

    out = _fused_call(rows3, W_lin.T, b_lin.reshape(1, D), l0, l1,
                      W_comb.T, b_comb.reshape(1, H), W2, b2.reshape(1, 1))
    return out.reshape(B)
